# baseline (speedup 1.0000x reference)
import functools

import jax
import jax.numpy as jnp
from jax import lax
from jax.experimental import pallas as pl
from jax.experimental.pallas import tpu as pltpu
from jax.experimental.pallas import tpu_sc as plsc

N = 10000
E = 320000
F_IN = 128
H = 32
G = 64

NC, NS, L = 2, 16, 16
NW = NC * NS
CHUNK = 128
EPT_CHUNKS = 80
E_PAD = NW * EPT_CHUNKS * CHUNK
K = 8
NGRP = EPT_CHUNKS // K
N_PAD = 10240
RPS = N_PAD // NS
DF = 16
R4 = N_PAD // 4
NR = N // 4

_mesh = plsc.VectorSubcoreMesh(
    core_axis_name="c", subcore_axis_name="s", num_cores=NC, num_subcores=NS)
_sc_params = pltpu.CompilerParams(use_tc_tiling_on_sc=False)
_sc_params_nl = pltpu.CompilerParams(use_tc_tiling_on_sc=False,
                                     needs_layout_passes=False)


@functools.partial(
    pl.kernel,
    out_type=[
        jax.ShapeDtypeStruct((N_PAD, H), jnp.float32),
        jax.ShapeDtypeStruct((N_PAD, H), jnp.float32),
    ],
    mesh=_mesh,
    scratch_types=[
        pltpu.VMEM((EPT_CHUNKS, CHUNK), jnp.int32),
        pltpu.VMEM((CHUNK, DF), jnp.float32),
        pltpu.VMEM((RPS, DF), jnp.float32),
        pltpu.VMEM((RPS,), jnp.float32),
        pltpu.VMEM((RPS, H), jnp.float32),
        pltpu.VMEM((RPS, H), jnp.float32),
        pltpu.VMEM_SHARED((N_PAD, DF), jnp.float32),
        pltpu.SemaphoreType.DMA,
    ],
    compiler_params=_sc_params_nl,
)
def _deg_sc(dstT_hbm, h1_hbm, dinv_hbm, hs1_hbm, dst_v, ones_v, cbuf, dtmp,
            rbuf, hbuf, acc_sh, sem):
    c = lax.axis_index("c")
    s = lax.axis_index("s")

    def _fill(val, i, carry):
        ones_v[i, pl.ds(0, 16)] = jnp.full((16,), val, jnp.float32)
        return carry

    lax.fori_loop(0, CHUNK, functools.partial(_fill, 0.0), 0)
    for k in range(RPS // CHUNK):
        pltpu.sync_copy(ones_v, acc_sh.at[pl.ds(s * RPS + k * CHUNK, CHUNK)])
    lax.fori_loop(0, CHUNK, functools.partial(_fill, 1.0), 0)
    plsc.subcore_barrier()

    def _body(g, carry):
        for b in range(8):
            pltpu.async_copy(ones_v, acc_sh.at[dst_v.at[g * 8 + b]], sem,
                             add=True)
        for b in range(8):
            pltpu.make_async_copy(ones_v, acc_sh.at[pl.ds(0, CHUNK)],
                                  sem).wait()
        return carry

    for wo in range(NC):
        pltpu.sync_copy(dstT_hbm.at[NC * s + wo], dst_v)
        lax.fori_loop(0, EPT_CHUNKS // 8, _body, 0)
    plsc.subcore_barrier()

    pltpu.sync_copy(acc_sh.at[pl.ds(s * RPS, RPS)], cbuf)
    zero16 = jnp.zeros((16,), jnp.int32)

    def _ck(k, carry):
        rows = lax.iota(jnp.int32, 16) + k * 16
        d16 = plsc.load_gather(cbuf, [rows, zero16]) + 1.0
        i32 = plsc.bitcast(d16, jnp.int32)
        i32 = jnp.full((16,), 0x5F3759DF, jnp.int32) - \
            lax.shift_right_logical(i32, 1)
        y = plsc.bitcast(i32, jnp.float32)
        for _ in range(3):
            y = y * (1.5 - 0.5 * d16 * y * y)
        dtmp[pl.ds(k * 16, 16)] = y
        return carry

    lax.fori_loop(0, RPS // 16, _ck, 0)

    pltpu.sync_copy(h1_hbm.at[pl.ds(s * RPS, RPS)], hbuf)

    def _br(r, carry):
        vv = plsc.load_gather(dtmp, [jnp.full((16,), 0, jnp.int32) + r])
        rbuf[r, pl.ds(0, 16)] = vv
        rbuf[r, pl.ds(16, 16)] = vv
        hbuf[r, pl.ds(0, 16)] = hbuf[r, pl.ds(0, 16)] * vv
        hbuf[r, pl.ds(16, 16)] = hbuf[r, pl.ds(16, 16)] * vv
        return carry

    lax.fori_loop(0, RPS, _br, 0)

    @pl.when(c == 0)
    def _():
        pltpu.sync_copy(rbuf, dinv_hbm.at[pl.ds(s * RPS, RPS)])
        pltpu.sync_copy(hbuf, hs1_hbm.at[pl.ds(s * RPS, RPS)])


@functools.partial(
    pl.kernel,
    out_type=jax.ShapeDtypeStruct((NC, N_PAD, H), jnp.float32),
    mesh=_mesh,
    scratch_types=[
        pltpu.VMEM((EPT_CHUNKS, CHUNK), jnp.int32),
        pltpu.VMEM((EPT_CHUNKS, CHUNK), jnp.int32),
        pltpu.VMEM((2 * K, CHUNK, H), jnp.float32),
        pltpu.VMEM_SHARED((N_PAD, H), jnp.float32),
        pltpu.VMEM_SHARED((N_PAD, H), jnp.float32),
        pltpu.SemaphoreType.DMA,
        pltpu.SemaphoreType.DMA,
    ],
    compiler_params=_sc_params,
)
def _agg_sc(hs_hbm, srcT_hbm, dstT_hbm, out_hbm, src_v, dst_v, buf_v, acc_sh,
            hs_sh, gsem, ssem):
    c = lax.axis_index("c")
    s = lax.axis_index("s")
    wid = s * NC + c
    pltpu.sync_copy(srcT_hbm.at[wid], src_v)
    pltpu.sync_copy(dstT_hbm.at[wid], dst_v)
    pltpu.sync_copy(hs_hbm.at[pl.ds(s * RPS, RPS)],
                    hs_sh.at[pl.ds(s * RPS, RPS)])

    def _zfill(i, carry):
        buf_v[0, i, pl.ds(0, 16)] = jnp.zeros((16,), jnp.float32)
        buf_v[0, i, pl.ds(16, 16)] = jnp.zeros((16,), jnp.float32)
        return carry

    lax.fori_loop(0, CHUNK, _zfill, 0)
    for k in range(RPS // CHUNK):
        pltpu.sync_copy(buf_v.at[0],
                        acc_sh.at[pl.ds(s * RPS + k * CHUNK, CHUNK)])
    plsc.subcore_barrier()

    for b in range(K):
        pltpu.async_copy(hs_sh.at[src_v.at[b]], buf_v.at[b], gsem)

    def _body(g, carry):
        base = g * K
        boff = (g % 2) * K
        nboff = K - boff
        for b in range(K):
            pltpu.make_async_copy(hs_hbm.at[pl.ds(0, CHUNK)],
                                  buf_v.at[b], gsem).wait()
        for b in range(K):
            pltpu.async_copy(buf_v.at[boff + b],
                             acc_sh.at[dst_v.at[base + b]], ssem, add=True)

        @pl.when(g < NGRP - 1)
        def _():
            for b in range(K):
                pltpu.async_copy(hs_sh.at[src_v.at[base + K + b]],
                                 buf_v.at[nboff + b], gsem)

        for b in range(K):
            pltpu.make_async_copy(buf_v.at[b], acc_sh.at[pl.ds(0, CHUNK)],
                                  ssem).wait()
        return carry

    lax.fori_loop(0, NGRP, _body, 0)
    plsc.subcore_barrier()
    pltpu.sync_copy(acc_sh.at[pl.ds(s * RPS, RPS)],
                    out_hbm.at[c, pl.ds(s * RPS, RPS)])


MB = 512
NMB = R4 // MB


def _mm1_body(x_ref, w_ref, out_ref):
    out_ref[...] = jnp.dot(x_ref[...], w_ref[...],
                           preferred_element_type=jnp.float32)


def _mm1_tc(x4, W1b):
    return pl.pallas_call(
        _mm1_body,
        grid=(NMB,),
        in_specs=[
            pl.BlockSpec((MB, 4 * F_IN), lambda i: (i, 0)),
            pl.BlockSpec((4 * F_IN, 128), lambda i: (0, 0)),
        ],
        out_specs=pl.BlockSpec((MB, 128), lambda i: (i, 0)),
        out_shape=jax.ShapeDtypeStruct((R4, 128), jnp.float32),
    )(x4, W1b)


def _fold4(v128):
    return (v128[:, 0:32] + v128[:, 32:64] + v128[:, 64:96] +
            v128[:, 96:128])


def _mid_body(aggp_ref, hs_ref, dinv_ref, b_ref, g_ref, be_ref, wn_ref,
              out_ref):
    dinv = dinv_ref[...]
    pre = dinv * (aggp_ref[0] + aggp_ref[1] + hs_ref[...]) + b_ref[...]
    t = jnp.tanh(pre)
    mask = (lax.broadcasted_iota(jnp.int32, (R4, 1), 0) < NR).astype(
        jnp.float32)
    tm = t * mask
    m32 = _fold4(jnp.sum(tm, axis=0, keepdims=True)) * (1.0 / N)
    q32 = _fold4(jnp.sum(tm * tm, axis=0, keepdims=True)) * (1.0 / N)
    var32 = q32 - m32 * m32
    r32 = lax.rsqrt(var32 + 1e-5)
    m128 = jnp.concatenate([m32] * 4, axis=1)
    r128 = jnp.concatenate([r32] * 4, axis=1)
    gr = g_ref[...] * r128
    c = jnp.dot(be_ref[...] - m128 * gr, wn_ref[...],
                preferred_element_type=jnp.float32)
    hn = jnp.dot(t * gr, wn_ref[...], preferred_element_type=jnp.float32) + c
    out_ref[...] = hn * dinv * mask


def _mid_tc(aggp, hsp, dinvp, b4, g4, be4, Wnb):
    return pl.pallas_call(
        _mid_body,
        out_shape=jax.ShapeDtypeStruct((R4, 128), jnp.float32),
    )(aggp, hsp, dinvp, b4, g4, be4, Wnb)


def _final_body(aggp_ref, hs_ref, dinv_ref, b_ref, batch_ref, wc_ref, bc_ref,
                out_ref):
    dinv = dinv_ref[...]
    pre = dinv * (aggp_ref[0] + aggp_ref[1] + hs_ref[...]) + b_ref[...]
    t = jnp.tanh(pre)
    gid = lax.broadcasted_iota(jnp.int32, (1, G), 1)
    ones = jnp.ones((R4, 1), jnp.float32)
    sums = jnp.zeros((G, H), jnp.float32)
    counts = jnp.zeros((G, 1), jnp.float32)
    for q in range(4):
        Pq = (batch_ref[:, q:q + 1] == gid).astype(jnp.float32)
        sums = sums + lax.dot_general(
            Pq, t[:, 32 * q:32 * q + 32], (((0,), (0,)), ((), ())),
            preferred_element_type=jnp.float32)
        counts = counts + lax.dot_general(
            Pq, ones, (((0,), (0,)), ((), ())),
            preferred_element_type=jnp.float32)
    pooled = sums / jnp.maximum(counts, 1.0)
    out_ref[...] = jnp.dot(pooled, wc_ref[...],
                           preferred_element_type=jnp.float32) + bc_ref[...]


def _final_tc(aggp, hsp, dinvp, b4, batch4, Wc, bc):
    return pl.pallas_call(
        _final_body,
        out_shape=jax.ShapeDtypeStruct((G, 1), jnp.float32),
    )(aggp, hsp, dinvp, b4, batch4, Wc, bc.reshape(1, 1))


def _tile4(v):
    return jnp.tile(v, 4).reshape(1, 128)


def kernel(x, edge_index, batch, W1, b1, g1, be1, W2, b2, g2, be2, W3, b3,
           Wc, bc):
    f32 = jnp.float32
    x_pad = jnp.pad(x, ((0, N_PAD - N), (0, 0)))
    x4 = x_pad.reshape(R4, 4 * F_IN)
    eye4 = jnp.eye(4, dtype=f32)
    W1b = jnp.kron(eye4, W1)
    W2b = jnp.kron(eye4, W2)
    W3b = jnp.kron(eye4, W3)
    pad_e = E_PAD - E
    src = jnp.concatenate(
        [edge_index[0], jnp.full((pad_e,), N, jnp.int32)])
    dst = jnp.concatenate(
        [edge_index[1], jnp.full((pad_e,), N, jnp.int32)])
    srcT = src.reshape(NW, EPT_CHUNKS, CHUNK)
    dstT = dst.reshape(NW, EPT_CHUNKS, CHUNK)
    batch4 = jnp.pad(batch, (0, N_PAD - N),
                     constant_values=G).reshape(R4, 4)

    h1p = _mm1_tc(x4, W1b)
    dinv_mat, hs1 = _deg_sc(dstT, h1p.reshape(N_PAD, H))
    dinvp = dinv_mat.reshape(R4, 128)
    hs1p = hs1.reshape(R4, 128)
    agg1 = _agg_sc(hs1, srcT, dstT)
    hs2p = _mid_tc(agg1.reshape(NC, R4, 128), hs1p, dinvp,
                   _tile4(b1), _tile4(g1), _tile4(be1), W2b)
    agg2 = _agg_sc(hs2p.reshape(N_PAD, H), srcT, dstT)
    hs3p = _mid_tc(agg2.reshape(NC, R4, 128), hs2p, dinvp,
                   _tile4(b2), _tile4(g2), _tile4(be2), W3b)
    agg3 = _agg_sc(hs3p.reshape(N_PAD, H), srcT, dstT)
    out = _final_tc(agg3.reshape(NC, R4, 128), hs3p, dinvp,
                    _tile4(b3), batch4, Wc, bc)
    return out


# speedup vs baseline: 1.0599x; 1.0599x over previous
import functools

import jax
import jax.numpy as jnp
from jax import lax
from jax.experimental import pallas as pl
from jax.experimental.pallas import tpu as pltpu
from jax.experimental.pallas import tpu_sc as plsc

N = 10000
E = 320000
F_IN = 128
H = 32
G = 64

NC, NS, L = 2, 16, 16
NW = NC * NS
CHUNK = 128
EPT_CHUNKS = 80
E_PAD = NW * EPT_CHUNKS * CHUNK
K = 8
NGRP = EPT_CHUNKS // K
N_PAD = 10240
RPS = N_PAD // NS
DF = 16
R4 = N_PAD // 4
NR = N // 4

_mesh = plsc.VectorSubcoreMesh(
    core_axis_name="c", subcore_axis_name="s", num_cores=NC, num_subcores=NS)
_sc_params = pltpu.CompilerParams(use_tc_tiling_on_sc=False)
_sc_params_nl = pltpu.CompilerParams(use_tc_tiling_on_sc=False,
                                     needs_layout_passes=False)


@functools.partial(
    pl.kernel,
    out_type=jax.ShapeDtypeStruct((N_PAD, H), jnp.float32),
    mesh=_mesh,
    scratch_types=[
        pltpu.VMEM((EPT_CHUNKS, CHUNK), jnp.int32),
        pltpu.VMEM((CHUNK, DF), jnp.float32),
        pltpu.VMEM((RPS, DF), jnp.float32),
        pltpu.VMEM((RPS,), jnp.float32),
        pltpu.VMEM((RPS, H), jnp.float32),
        pltpu.VMEM_SHARED((N_PAD, DF), jnp.float32),
        pltpu.SemaphoreType.DMA,
    ],
    compiler_params=_sc_params_nl,
)
def _deg_sc(dstT_hbm, dinv_hbm, dst_v, ones_v, cbuf, dtmp, rbuf, acc_sh, sem):
    c = lax.axis_index("c")
    s = lax.axis_index("s")

    def _fill(val, i, carry):
        ones_v[i, pl.ds(0, 16)] = jnp.full((16,), val, jnp.float32)
        return carry

    lax.fori_loop(0, CHUNK, functools.partial(_fill, 0.0), 0)
    for k in range(RPS // CHUNK):
        pltpu.sync_copy(ones_v, acc_sh.at[pl.ds(s * RPS + k * CHUNK, CHUNK)])
    lax.fori_loop(0, CHUNK, functools.partial(_fill, 1.0), 0)
    plsc.subcore_barrier()

    def _body(g, carry):
        for b in range(8):
            pltpu.async_copy(ones_v, acc_sh.at[dst_v.at[g * 8 + b]], sem,
                             add=True)
        for b in range(8):
            pltpu.make_async_copy(ones_v, acc_sh.at[pl.ds(0, CHUNK)],
                                  sem).wait()
        return carry

    for wo in range(NC):
        pltpu.sync_copy(dstT_hbm.at[NC * s + wo], dst_v)
        lax.fori_loop(0, EPT_CHUNKS // 8, _body, 0)
    plsc.subcore_barrier()

    pltpu.sync_copy(acc_sh.at[pl.ds(s * RPS, RPS)], cbuf)
    zero16 = jnp.zeros((16,), jnp.int32)

    def _ck(k, carry):
        rows = lax.iota(jnp.int32, 16) + k * 16
        d16 = plsc.load_gather(cbuf, [rows, zero16]) + 1.0
        i32 = plsc.bitcast(d16, jnp.int32)
        i32 = jnp.full((16,), 0x5F3759DF, jnp.int32) - \
            lax.shift_right_logical(i32, 1)
        y = plsc.bitcast(i32, jnp.float32)
        for _ in range(3):
            y = y * (1.5 - 0.5 * d16 * y * y)
        dtmp[pl.ds(k * 16, 16)] = y
        return carry

    lax.fori_loop(0, RPS // 16, _ck, 0)

    def _br(r, carry):
        vv = plsc.load_gather(dtmp, [jnp.full((16,), 0, jnp.int32) + r])
        rbuf[r, pl.ds(0, 16)] = vv
        rbuf[r, pl.ds(16, 16)] = vv
        return carry

    lax.fori_loop(0, RPS, _br, 0)

    @pl.when(c == 0)
    def _():
        pltpu.sync_copy(rbuf, dinv_hbm.at[pl.ds(s * RPS, RPS)])


@functools.partial(
    pl.kernel,
    out_type=jax.ShapeDtypeStruct((NC, N_PAD, H), jnp.float32),
    mesh=_mesh,
    scratch_types=[
        pltpu.VMEM((EPT_CHUNKS, CHUNK), jnp.int32),
        pltpu.VMEM((EPT_CHUNKS, CHUNK), jnp.int32),
        pltpu.VMEM((2 * K, CHUNK, H), jnp.float32),
        pltpu.VMEM_SHARED((N_PAD, H), jnp.float32),
        pltpu.VMEM_SHARED((N_PAD, H), jnp.float32),
        pltpu.SemaphoreType.DMA,
        pltpu.SemaphoreType.DMA,
    ],
    compiler_params=_sc_params,
)
def _agg_sc(hs_hbm, srcT_hbm, dstT_hbm, out_hbm, src_v, dst_v, buf_v, acc_sh,
            hs_sh, gsem, ssem):
    c = lax.axis_index("c")
    s = lax.axis_index("s")
    wid = s * NC + c
    pltpu.sync_copy(srcT_hbm.at[wid], src_v)
    pltpu.sync_copy(dstT_hbm.at[wid], dst_v)
    pltpu.sync_copy(hs_hbm.at[pl.ds(s * RPS, RPS)],
                    hs_sh.at[pl.ds(s * RPS, RPS)])

    def _zfill(i, carry):
        buf_v[0, i, pl.ds(0, 16)] = jnp.zeros((16,), jnp.float32)
        buf_v[0, i, pl.ds(16, 16)] = jnp.zeros((16,), jnp.float32)
        return carry

    lax.fori_loop(0, CHUNK, _zfill, 0)
    for k in range(RPS // CHUNK):
        pltpu.sync_copy(buf_v.at[0],
                        acc_sh.at[pl.ds(s * RPS + k * CHUNK, CHUNK)])
    plsc.subcore_barrier()

    for b in range(K):
        pltpu.async_copy(hs_sh.at[src_v.at[b]], buf_v.at[b], gsem)

    def _body(g, carry):
        base = g * K
        boff = (g % 2) * K
        nboff = K - boff
        for b in range(K):
            pltpu.make_async_copy(hs_hbm.at[pl.ds(0, CHUNK)],
                                  buf_v.at[b], gsem).wait()
        for b in range(K):
            pltpu.async_copy(buf_v.at[boff + b],
                             acc_sh.at[dst_v.at[base + b]], ssem, add=True)

        @pl.when(g < NGRP - 1)
        def _():
            for b in range(K):
                pltpu.async_copy(hs_sh.at[src_v.at[base + K + b]],
                                 buf_v.at[nboff + b], gsem)

        for b in range(K):
            pltpu.make_async_copy(buf_v.at[b], acc_sh.at[pl.ds(0, CHUNK)],
                                  ssem).wait()
        return carry

    lax.fori_loop(0, NGRP, _body, 0)
    plsc.subcore_barrier()
    pltpu.sync_copy(acc_sh.at[pl.ds(s * RPS, RPS)],
                    out_hbm.at[c, pl.ds(s * RPS, RPS)])


MB = 512
NMB = R4 // MB


def _mm1_body(x_ref, w_ref, out_ref):
    out_ref[...] = jnp.dot(x_ref[...], w_ref[...],
                           preferred_element_type=jnp.float32)


def _mm1_tc(x4, W1b):
    return pl.pallas_call(
        _mm1_body,
        grid=(NMB,),
        in_specs=[
            pl.BlockSpec((MB, 4 * F_IN), lambda i: (i, 0)),
            pl.BlockSpec((4 * F_IN, 128), lambda i: (0, 0)),
        ],
        out_specs=pl.BlockSpec((MB, 128), lambda i: (i, 0)),
        out_shape=jax.ShapeDtypeStruct((R4, 128), jnp.float32),
    )(x4, W1b)


def _scale_body(h_ref, dinv_ref, out_ref):
    out_ref[...] = h_ref[...] * dinv_ref[...]


def _scale_tc(h1p, dinvp):
    return pl.pallas_call(
        _scale_body,
        grid=(NMB,),
        in_specs=[
            pl.BlockSpec((MB, 128), lambda i: (i, 0)),
            pl.BlockSpec((MB, 128), lambda i: (i, 0)),
        ],
        out_specs=pl.BlockSpec((MB, 128), lambda i: (i, 0)),
        out_shape=jax.ShapeDtypeStruct((R4, 128), jnp.float32),
    )(h1p, dinvp)


def _fold4(v128):
    return (v128[:, 0:32] + v128[:, 32:64] + v128[:, 64:96] +
            v128[:, 96:128])


def _mid_body(aggp_ref, hs_ref, dinv_ref, b_ref, g_ref, be_ref, wn_ref,
              out_ref):
    dinv = dinv_ref[...]
    pre = dinv * (aggp_ref[0] + aggp_ref[1] + hs_ref[...]) + b_ref[...]
    t = jnp.tanh(pre)
    mask = (lax.broadcasted_iota(jnp.int32, (R4, 1), 0) < NR).astype(
        jnp.float32)
    tm = t * mask
    m32 = _fold4(jnp.sum(tm, axis=0, keepdims=True)) * (1.0 / N)
    q32 = _fold4(jnp.sum(tm * tm, axis=0, keepdims=True)) * (1.0 / N)
    var32 = q32 - m32 * m32
    r32 = lax.rsqrt(var32 + 1e-5)
    m128 = jnp.concatenate([m32] * 4, axis=1)
    r128 = jnp.concatenate([r32] * 4, axis=1)
    gr = g_ref[...] * r128
    c = jnp.dot(be_ref[...] - m128 * gr, wn_ref[...],
                preferred_element_type=jnp.float32)
    hn = jnp.dot(t * gr, wn_ref[...], preferred_element_type=jnp.float32) + c
    out_ref[...] = hn * dinv * mask


def _mid_tc(aggp, hsp, dinvp, b4, g4, be4, Wnb):
    return pl.pallas_call(
        _mid_body,
        out_shape=jax.ShapeDtypeStruct((R4, 128), jnp.float32),
    )(aggp, hsp, dinvp, b4, g4, be4, Wnb)


def _final_body(aggp_ref, hs_ref, dinv_ref, b_ref, batch_ref, wc_ref, bc_ref,
                out_ref):
    dinv = dinv_ref[...]
    pre = dinv * (aggp_ref[0] + aggp_ref[1] + hs_ref[...]) + b_ref[...]
    t = jnp.tanh(pre)
    gid = lax.broadcasted_iota(jnp.int32, (1, G), 1)
    ones = jnp.ones((R4, 1), jnp.float32)
    sums = jnp.zeros((G, H), jnp.float32)
    counts = jnp.zeros((G, 1), jnp.float32)
    for q in range(4):
        Pq = (batch_ref[:, q:q + 1] == gid).astype(jnp.float32)
        sums = sums + lax.dot_general(
            Pq, t[:, 32 * q:32 * q + 32], (((0,), (0,)), ((), ())),
            preferred_element_type=jnp.float32)
        counts = counts + lax.dot_general(
            Pq, ones, (((0,), (0,)), ((), ())),
            preferred_element_type=jnp.float32)
    pooled = sums / jnp.maximum(counts, 1.0)
    out_ref[...] = jnp.dot(pooled, wc_ref[...],
                           preferred_element_type=jnp.float32) + bc_ref[...]


def _final_tc(aggp, hsp, dinvp, b4, batch4, Wc, bc):
    return pl.pallas_call(
        _final_body,
        out_shape=jax.ShapeDtypeStruct((G, 1), jnp.float32),
    )(aggp, hsp, dinvp, b4, batch4, Wc, bc.reshape(1, 1))


def _tile4(v):
    return jnp.tile(v, 4).reshape(1, 128)


def kernel(x, edge_index, batch, W1, b1, g1, be1, W2, b2, g2, be2, W3, b3,
           Wc, bc):
    f32 = jnp.float32
    x_pad = jnp.pad(x, ((0, N_PAD - N), (0, 0)))
    x4 = x_pad.reshape(R4, 4 * F_IN)
    eye4 = jnp.eye(4, dtype=f32)
    W1b = jnp.kron(eye4, W1)
    W2b = jnp.kron(eye4, W2)
    W3b = jnp.kron(eye4, W3)
    pad_e = E_PAD - E
    src = jnp.concatenate(
        [edge_index[0], jnp.full((pad_e,), N, jnp.int32)])
    dst = jnp.concatenate(
        [edge_index[1], jnp.full((pad_e,), N, jnp.int32)])
    srcT = src.reshape(NW, EPT_CHUNKS, CHUNK)
    dstT = dst.reshape(NW, EPT_CHUNKS, CHUNK)
    batch4 = jnp.pad(batch, (0, N_PAD - N),
                     constant_values=G).reshape(R4, 4)

    dinvp = _deg_sc(dstT).reshape(R4, 128)
    h1p = _mm1_tc(x4, W1b)
    hs1p = _scale_tc(h1p, dinvp)
    agg1 = _agg_sc(hs1p.reshape(N_PAD, H), srcT, dstT)
    hs2p = _mid_tc(agg1.reshape(NC, R4, 128), hs1p, dinvp,
                   _tile4(b1), _tile4(g1), _tile4(be1), W2b)
    agg2 = _agg_sc(hs2p.reshape(N_PAD, H), srcT, dstT)
    hs3p = _mid_tc(agg2.reshape(NC, R4, 128), hs2p, dinvp,
                   _tile4(b2), _tile4(g2), _tile4(be2), W3b)
    agg3 = _agg_sc(hs3p.reshape(N_PAD, H), srcT, dstT)
    out = _final_tc(agg3.reshape(NC, R4, 128), hs3p, dinvp,
                    _tile4(b3), batch4, Wc, bc)
    return out
